# baseline (speedup 1.0000x reference)
import jax
import jax.numpy as jnp
from jax import lax
from jax.experimental import pallas as pl

C = 384
H = 8
HD = 48
NW = 64
WS = 32
TOPK = 4


def _qkv_lif_kernel(x_ref, w_ref, b_ref, q_ref, k_ref, v_ref):
    x = x_ref[...]
    acc = jnp.dot(x, w_ref[...], preferred_element_type=jnp.float32) + b_ref[...]
    s = (acc >= 2.0).astype(jnp.float32)
    q_ref[...] = s[:, :C]
    k_ref[...] = s[:, C:2 * C]
    v_ref[...] = s[:, 2 * C:]


def _routing_kernel(xr_ref, m_ref):
    xs = xr_ref[0]
    feat = jnp.sum(xs, axis=1) * (1.0 / WS)
    scores = lax.dot_general(
        feat, feat, (((1,), (1,)), ((), ())),
        preferred_element_type=jnp.float32) * (HD ** -0.5)
    iota = lax.broadcasted_iota(jnp.int32, (NW, NW), 1)
    mask = jnp.zeros((NW, NW), jnp.float32)
    s = scores
    for _ in range(TOPK):
        mx = jnp.max(s, axis=1, keepdims=True)
        cand = jnp.where(s >= mx, iota, jnp.int32(2 ** 30))
        mi = jnp.min(cand, axis=1, keepdims=True)
        sel = iota == mi
        mask = jnp.where(sel, 1.0, mask)
        s = jnp.where(sel, -jnp.inf, s)
    m_ref[0] = mask


def _kvw_kernel(k_ref, v_ref, kvw_ref, ksum_ref):
    k = k_ref[0]
    v = v_ref[0]
    outs = []
    for h in range(H):
        kh = k[:, h * HD:(h + 1) * HD]
        vh = v[:, h * HD:(h + 1) * HD]
        outs.append(lax.dot_general(
            kh, vh, (((0,), (0,)), ((), ())),
            preferred_element_type=jnp.float32))
    kvw_ref[0] = jnp.concatenate(outs, axis=1)
    ksum_ref[0] = jnp.sum(k, axis=0, keepdims=True)


def _route_kv_kernel(m_ref, kvw_ref, ksum_ref, kvr_ref, ksr_ref):
    m = m_ref[0]
    kvr_ref[0, 0] = jnp.dot(m, kvw_ref[0, 0], preferred_element_type=jnp.float32)
    ksr_ref[0, 0] = jnp.dot(m, ksum_ref[0, 0], preferred_element_type=jnp.float32)


def _attn_proj_kernel(q_ref, kvr_ref, ksr_ref, wp_ref, bp_ref, o_ref):
    q = q_ref[0]
    kvr = kvr_ref[0]
    ks = ksr_ref[0]
    outs = []
    for h in range(H):
        sl = slice(h * HD, (h + 1) * HD)
        qh = q[:, sl]
        oh = jnp.dot(qh, kvr[:, sl], preferred_element_type=jnp.float32)
        dh = jnp.sum(qh * ks[:, sl], axis=1, keepdims=True)
        outs.append(oh / (dh + 1e-6))
    o = jnp.concatenate(outs, axis=1)
    o_ref[0] = jnp.dot(o, wp_ref[...], preferred_element_type=jnp.float32) + bp_ref[...]


def kernel(x, W_qkv, b_qkv, W_proj, b_proj):
    T, B, Lt, Lh, Lw, _ = x.shape
    wt, wh, ww = 4, 4, 4
    nt, nh, nd = Lt // wt, Lh // wh, Lw // ww
    TBN = T * B * NW
    NTOK = TBN * WS

    xw = x.reshape(T, B, wt, nt, wh, nh, ww, nd, C)
    xw = jnp.transpose(xw, (0, 1, 2, 4, 6, 3, 5, 7, 8))
    x_win = xw.reshape(T, B, NW, WS, C)
    x_tok = x_win.reshape(NTOK, C)
    xr = jnp.transpose(x_win, (1, 2, 0, 3, 4)).reshape(B, NW, T * WS, C)

    Wt_qkv = W_qkv.T
    bq = b_qkv.reshape(1, 3 * C)

    nblk = 16
    q_s, k_s, v_s = pl.pallas_call(
        _qkv_lif_kernel,
        grid=(nblk,),
        in_specs=[pl.BlockSpec((NTOK // nblk, C), lambda i: (i, 0)),
                  pl.BlockSpec((C, 3 * C), lambda i: (0, 0)),
                  pl.BlockSpec((1, 3 * C), lambda i: (0, 0))],
        out_specs=[pl.BlockSpec((NTOK // nblk, C), lambda i: (i, 0))] * 3,
        out_shape=[jax.ShapeDtypeStruct((NTOK, C), jnp.float32)] * 3,
    )(x_tok, Wt_qkv, bq)

    mask = pl.pallas_call(
        _routing_kernel,
        grid=(B,),
        in_specs=[pl.BlockSpec((1, NW, T * WS, C), lambda b: (b, 0, 0, 0))],
        out_specs=pl.BlockSpec((1, NW, NW), lambda b: (b, 0, 0)),
        out_shape=jax.ShapeDtypeStruct((B, NW, NW), jnp.float32),
    )(xr)

    k3 = k_s.reshape(TBN, WS, C)
    v3 = v_s.reshape(TBN, WS, C)
    kvw, ksum = pl.pallas_call(
        _kvw_kernel,
        grid=(TBN,),
        in_specs=[pl.BlockSpec((1, WS, C), lambda i: (i, 0, 0)),
                  pl.BlockSpec((1, WS, C), lambda i: (i, 0, 0))],
        out_specs=[pl.BlockSpec((1, HD, C), lambda i: (i, 0, 0)),
                   pl.BlockSpec((1, 1, C), lambda i: (i, 0, 0))],
        out_shape=[jax.ShapeDtypeStruct((TBN, HD, C), jnp.float32),
                   jax.ShapeDtypeStruct((TBN, 1, C), jnp.float32)],
    )(k3, v3)

    kvw4 = kvw.reshape(T, B, NW, HD * C)
    ksum4 = ksum.reshape(T, B, NW, C)
    kv_r, ks_r = pl.pallas_call(
        _route_kv_kernel,
        grid=(T, B),
        in_specs=[pl.BlockSpec((1, NW, NW), lambda t, b: (b, 0, 0)),
                  pl.BlockSpec((1, 1, NW, HD * C), lambda t, b: (t, b, 0, 0)),
                  pl.BlockSpec((1, 1, NW, C), lambda t, b: (t, b, 0, 0))],
        out_specs=[pl.BlockSpec((1, 1, NW, HD * C), lambda t, b: (t, b, 0, 0)),
                   pl.BlockSpec((1, 1, NW, C), lambda t, b: (t, b, 0, 0))],
        out_shape=[jax.ShapeDtypeStruct((T, B, NW, HD * C), jnp.float32),
                   jax.ShapeDtypeStruct((T, B, NW, C), jnp.float32)],
    )(mask, kvw4, ksum4)

    q3 = q_s.reshape(TBN, WS, C)
    kv_r3 = kv_r.reshape(TBN, HD, C)
    ks_r3 = ks_r.reshape(TBN, 1, C)
    out = pl.pallas_call(
        _attn_proj_kernel,
        grid=(TBN,),
        in_specs=[pl.BlockSpec((1, WS, C), lambda i: (i, 0, 0)),
                  pl.BlockSpec((1, HD, C), lambda i: (i, 0, 0)),
                  pl.BlockSpec((1, 1, C), lambda i: (i, 0, 0)),
                  pl.BlockSpec((C, C), lambda i: (0, 0)),
                  pl.BlockSpec((1, C), lambda i: (0, 0))],
        out_specs=pl.BlockSpec((1, WS, C), lambda i: (i, 0, 0)),
        out_shape=jax.ShapeDtypeStruct((TBN, WS, C), jnp.float32),
    )(q3, kv_r3, ks_r3, W_proj.T, b_proj.reshape(1, C))

    out = out.reshape(T, B, wt, wh, ww, nt, nh, nd, C)
    out = jnp.transpose(out, (0, 1, 2, 5, 3, 6, 4, 7, 8))
    return out.reshape(T, B, Lt, Lh, Lw, C)


# speedup vs baseline: 2.1172x; 2.1172x over previous
import jax
import jax.numpy as jnp
from jax import lax
from jax.experimental import pallas as pl

C = 384
H = 8
HD = 48
NW = 64
WS = 32
TOPK = 4


def _qkv_lif_kernel(x_ref, w_ref, b_ref, q_ref, k_ref, v_ref):
    x = x_ref[...]
    acc = jnp.dot(x, w_ref[...], preferred_element_type=jnp.float32) + b_ref[...]
    s = (acc >= 2.0).astype(jnp.float32)
    q_ref[...] = s[:, :C]
    k_ref[...] = s[:, C:2 * C]
    v_ref[...] = s[:, 2 * C:]


def _routing_kernel(xr_ref, m_ref):
    xs = xr_ref[:, 0]
    feat = jnp.sum(jnp.sum(xs, axis=0), axis=1) * (1.0 / WS)
    scores = lax.dot_general(
        feat, feat, (((1,), (1,)), ((), ())),
        preferred_element_type=jnp.float32) * (HD ** -0.5)
    iota = lax.broadcasted_iota(jnp.int32, (NW, NW), 1)
    mask = jnp.zeros((NW, NW), jnp.float32)
    s = scores
    for _ in range(TOPK):
        mx = jnp.max(s, axis=1, keepdims=True)
        cand = jnp.where(s >= mx, iota, jnp.int32(2 ** 30))
        mi = jnp.min(cand, axis=1, keepdims=True)
        sel = iota == mi
        mask = jnp.where(sel, 1.0, mask)
        s = jnp.where(sel, -jnp.inf, s)
    m_ref[0] = mask


WB = 8


def _kvw_kernel(k_ref, v_ref, kvw_ref, ksum_ref):
    for w in range(WB):
        k = k_ref[w]
        v = v_ref[w]
        kvf = lax.dot_general(k, v, (((0,), (0,)), ((), ())),
                              preferred_element_type=jnp.float32)
        pieces = [lax.slice(kvf, (h * HD, h * HD), ((h + 1) * HD, (h + 1) * HD))
                  for h in range(H)]
        kvw_ref[w] = jnp.concatenate(pieces, axis=1)
        ksum_ref[w] = jnp.sum(k, axis=0, keepdims=True)


def _route_kv_kernel(m_ref, kvw_ref, ksum_ref, kvr_ref, ksr_ref):
    m = m_ref[0]
    kvr_ref[0, 0] = jnp.dot(m, kvw_ref[0, 0], preferred_element_type=jnp.float32)
    ksr_ref[0, 0] = jnp.dot(m, ksum_ref[0, 0], preferred_element_type=jnp.float32)


def _attn_proj_kernel(q_ref, kvr_ref, ksr_ref, bdm_ref, wp_ref, bp_ref, o_ref):
    bdm = bdm_ref[...]
    rows = []
    for w in range(WB):
        q = q_ref[w]
        kvr = kvr_ref[w]
        ks = ksr_ref[w]
        bd = jnp.concatenate([kvr] * H, axis=0) * bdm
        numer = jnp.dot(q, bd, preferred_element_type=jnp.float32)
        drep = jnp.dot(q * ks, bdm, preferred_element_type=jnp.float32)
        rows.append(numer / (drep + 1e-6))
    o = jnp.concatenate(rows, axis=0)
    o = jnp.dot(o, wp_ref[...], preferred_element_type=jnp.float32) + bp_ref[...]
    for w in range(WB):
        o_ref[w] = o[w * WS:(w + 1) * WS, :]


def kernel(x, W_qkv, b_qkv, W_proj, b_proj):
    T, B, Lt, Lh, Lw, _ = x.shape
    wt, wh, ww = 4, 4, 4
    nt, nh, nd = Lt // wt, Lh // wh, Lw // ww
    TBN = T * B * NW
    NTOK = TBN * WS

    xw = x.reshape(T, B, wt, nt, wh, nh, ww, nd, C)
    xw = jnp.transpose(xw, (0, 1, 2, 4, 6, 3, 5, 7, 8))
    x_win = xw.reshape(T, B, NW, WS, C)
    x_tok = x_win.reshape(NTOK, C)

    Wt_qkv = W_qkv.T
    bq = b_qkv.reshape(1, 3 * C)

    nblk = 16
    q_s, k_s, v_s = pl.pallas_call(
        _qkv_lif_kernel,
        grid=(nblk,),
        in_specs=[pl.BlockSpec((NTOK // nblk, C), lambda i: (i, 0)),
                  pl.BlockSpec((C, 3 * C), lambda i: (0, 0)),
                  pl.BlockSpec((1, 3 * C), lambda i: (0, 0))],
        out_specs=[pl.BlockSpec((NTOK // nblk, C), lambda i: (i, 0))] * 3,
        out_shape=[jax.ShapeDtypeStruct((NTOK, C), jnp.float32)] * 3,
    )(x_tok, Wt_qkv, bq)

    mask = pl.pallas_call(
        _routing_kernel,
        grid=(B,),
        in_specs=[pl.BlockSpec((T, 1, NW, WS, C), lambda b: (0, b, 0, 0, 0))],
        out_specs=pl.BlockSpec((1, NW, NW), lambda b: (b, 0, 0)),
        out_shape=jax.ShapeDtypeStruct((B, NW, NW), jnp.float32),
    )(x_win)

    k3 = k_s.reshape(TBN, WS, C)
    v3 = v_s.reshape(TBN, WS, C)
    kvw, ksum = pl.pallas_call(
        _kvw_kernel,
        grid=(TBN // WB,),
        in_specs=[pl.BlockSpec((WB, WS, C), lambda i: (i, 0, 0)),
                  pl.BlockSpec((WB, WS, C), lambda i: (i, 0, 0))],
        out_specs=[pl.BlockSpec((WB, HD, C), lambda i: (i, 0, 0)),
                   pl.BlockSpec((WB, 1, C), lambda i: (i, 0, 0))],
        out_shape=[jax.ShapeDtypeStruct((TBN, HD, C), jnp.float32),
                   jax.ShapeDtypeStruct((TBN, 1, C), jnp.float32)],
    )(k3, v3)

    kvw4 = kvw.reshape(T, B, NW, HD * C)
    ksum4 = ksum.reshape(T, B, NW, C)
    kv_r, ks_r = pl.pallas_call(
        _route_kv_kernel,
        grid=(T, B),
        in_specs=[pl.BlockSpec((1, NW, NW), lambda t, b: (b, 0, 0)),
                  pl.BlockSpec((1, 1, NW, HD * C), lambda t, b: (t, b, 0, 0)),
                  pl.BlockSpec((1, 1, NW, C), lambda t, b: (t, b, 0, 0))],
        out_specs=[pl.BlockSpec((1, 1, NW, HD * C), lambda t, b: (t, b, 0, 0)),
                   pl.BlockSpec((1, 1, NW, C), lambda t, b: (t, b, 0, 0))],
        out_shape=[jax.ShapeDtypeStruct((T, B, NW, HD * C), jnp.float32),
                   jax.ShapeDtypeStruct((T, B, NW, C), jnp.float32)],
    )(mask, kvw4, ksum4)

    q3 = q_s.reshape(TBN, WS, C)
    kv_r3 = kv_r.reshape(TBN, HD, C)
    ks_r3 = ks_r.reshape(TBN, 1, C)
    hid = jnp.arange(C, dtype=jnp.int32) // HD
    bdmask = (hid[:, None] == hid[None, :]).astype(jnp.float32)
    out = pl.pallas_call(
        _attn_proj_kernel,
        grid=(TBN // WB,),
        in_specs=[pl.BlockSpec((WB, WS, C), lambda i: (i, 0, 0)),
                  pl.BlockSpec((WB, HD, C), lambda i: (i, 0, 0)),
                  pl.BlockSpec((WB, 1, C), lambda i: (i, 0, 0)),
                  pl.BlockSpec((C, C), lambda i: (0, 0)),
                  pl.BlockSpec((C, C), lambda i: (0, 0)),
                  pl.BlockSpec((1, C), lambda i: (0, 0))],
        out_specs=pl.BlockSpec((WB, WS, C), lambda i: (i, 0, 0)),
        out_shape=jax.ShapeDtypeStruct((TBN, WS, C), jnp.float32),
    )(q3, kv_r3, ks_r3, bdmask, W_proj.T, b_proj.reshape(1, C))

    out = out.reshape(T, B, wt, wh, ww, nt, nh, nd, C)
    out = jnp.transpose(out, (0, 1, 2, 5, 3, 6, 4, 7, 8))
    return out.reshape(T, B, Lt, Lh, Lw, C)


# speedup vs baseline: 2.5115x; 1.1862x over previous
import jax
import jax.numpy as jnp
from jax import lax
from jax.experimental import pallas as pl

C = 384
H = 8
HD = 48
NW = 64
WS = 32
TOPK = 4


CHW = 32


def _qkv_kvw_kernel(x_ref, w_ref, b_ref, q_ref, kvw_ref, ksum_ref, feat_ref):
    x = x_ref[...]
    acc = jnp.dot(x, w_ref[...], preferred_element_type=jnp.float32) + b_ref[...]
    s = (acc >= 2.0).astype(jnp.float32)
    q_ref[...] = s[:, :C]
    k = s[:, C:2 * C]
    v = s[:, 2 * C:]
    for w in range(CHW):
        rows = slice(w * WS, (w + 1) * WS)
        kw = k[rows]
        vw = v[rows]
        kvf = lax.dot_general(kw, vw, (((0,), (0,)), ((), ())),
                              preferred_element_type=jnp.float32)
        pieces = [lax.slice(kvf, (h * HD, h * HD), ((h + 1) * HD, (h + 1) * HD))
                  for h in range(H)]
        kvw_ref[w] = jnp.concatenate(pieces, axis=1)
        ksum_ref[w] = jnp.sum(kw, axis=0, keepdims=True)
        feat_ref[w] = jnp.sum(x[rows], axis=0, keepdims=True)


def _routing_kernel(xr_ref, m_ref):
    xs = xr_ref[:, 0]
    feat = jnp.sum(xs, axis=0) * (1.0 / WS)
    scores = lax.dot_general(
        feat, feat, (((1,), (1,)), ((), ())),
        preferred_element_type=jnp.float32) * (HD ** -0.5)
    iota = lax.broadcasted_iota(jnp.int32, (NW, NW), 1)
    mask = jnp.zeros((NW, NW), jnp.float32)
    s = scores
    for _ in range(TOPK):
        mx = jnp.max(s, axis=1, keepdims=True)
        cand = jnp.where(s >= mx, iota, jnp.int32(2 ** 30))
        mi = jnp.min(cand, axis=1, keepdims=True)
        sel = iota == mi
        mask = jnp.where(sel, 1.0, mask)
        s = jnp.where(sel, -jnp.inf, s)
    m_ref[0] = mask


WB = 8


def _route_kv_kernel(m_ref, kvw_ref, ksum_ref, kvr_ref, ksr_ref):
    m = m_ref[0]
    kvr_ref[0, 0] = jnp.dot(m, kvw_ref[0, 0], preferred_element_type=jnp.float32)
    ksr_ref[0, 0] = jnp.dot(m, ksum_ref[0, 0], preferred_element_type=jnp.float32)


def _attn_proj_kernel(q_ref, kvr_ref, ksr_ref, bdm_ref, wp_ref, bp_ref, o_ref):
    bdm = bdm_ref[...]
    rows = []
    for w in range(WB):
        q = q_ref[w]
        kvr = kvr_ref[w]
        ks = ksr_ref[w]
        bd = jnp.concatenate([kvr] * H, axis=0) * bdm
        numer = jnp.dot(q, bd, preferred_element_type=jnp.float32)
        drep = jnp.dot(q * ks, bdm, preferred_element_type=jnp.float32)
        rows.append(numer / (drep + 1e-6))
    o = jnp.concatenate(rows, axis=0)
    o = jnp.dot(o, wp_ref[...], preferred_element_type=jnp.float32) + bp_ref[...]
    for w in range(WB):
        o_ref[w] = o[w * WS:(w + 1) * WS, :]


def kernel(x, W_qkv, b_qkv, W_proj, b_proj):
    T, B, Lt, Lh, Lw, _ = x.shape
    wt, wh, ww = 4, 4, 4
    nt, nh, nd = Lt // wt, Lh // wh, Lw // ww
    TBN = T * B * NW
    NTOK = TBN * WS

    xw = x.reshape(T, B, wt, nt, wh, nh, ww, nd, C)
    xw = jnp.transpose(xw, (0, 1, 2, 4, 6, 3, 5, 7, 8))
    x_win = xw.reshape(T, B, NW, WS, C)
    x_tok = x_win.reshape(NTOK, C)

    Wt_qkv = W_qkv.T
    bq = b_qkv.reshape(1, 3 * C)

    nblk = TBN // CHW
    q_s, kvw, ksum, feats = pl.pallas_call(
        _qkv_kvw_kernel,
        grid=(nblk,),
        in_specs=[pl.BlockSpec((NTOK // nblk, C), lambda i: (i, 0)),
                  pl.BlockSpec((C, 3 * C), lambda i: (0, 0)),
                  pl.BlockSpec((1, 3 * C), lambda i: (0, 0))],
        out_specs=[pl.BlockSpec((NTOK // nblk, C), lambda i: (i, 0)),
                   pl.BlockSpec((CHW, HD, C), lambda i: (i, 0, 0)),
                   pl.BlockSpec((CHW, 1, C), lambda i: (i, 0, 0)),
                   pl.BlockSpec((CHW, 1, C), lambda i: (i, 0, 0))],
        out_shape=[jax.ShapeDtypeStruct((NTOK, C), jnp.float32),
                   jax.ShapeDtypeStruct((TBN, HD, C), jnp.float32),
                   jax.ShapeDtypeStruct((TBN, 1, C), jnp.float32),
                   jax.ShapeDtypeStruct((TBN, 1, C), jnp.float32)],
    )(x_tok, Wt_qkv, bq)

    feats4 = feats.reshape(T, B, NW, C)
    mask = pl.pallas_call(
        _routing_kernel,
        grid=(B,),
        in_specs=[pl.BlockSpec((T, 1, NW, C), lambda b: (0, b, 0, 0))],
        out_specs=pl.BlockSpec((1, NW, NW), lambda b: (b, 0, 0)),
        out_shape=jax.ShapeDtypeStruct((B, NW, NW), jnp.float32),
    )(feats4)

    kvw4 = kvw.reshape(T, B, NW, HD * C)
    ksum4 = ksum.reshape(T, B, NW, C)
    kv_r, ks_r = pl.pallas_call(
        _route_kv_kernel,
        grid=(T, B),
        in_specs=[pl.BlockSpec((1, NW, NW), lambda t, b: (b, 0, 0)),
                  pl.BlockSpec((1, 1, NW, HD * C), lambda t, b: (t, b, 0, 0)),
                  pl.BlockSpec((1, 1, NW, C), lambda t, b: (t, b, 0, 0))],
        out_specs=[pl.BlockSpec((1, 1, NW, HD * C), lambda t, b: (t, b, 0, 0)),
                   pl.BlockSpec((1, 1, NW, C), lambda t, b: (t, b, 0, 0))],
        out_shape=[jax.ShapeDtypeStruct((T, B, NW, HD * C), jnp.float32),
                   jax.ShapeDtypeStruct((T, B, NW, C), jnp.float32)],
    )(mask, kvw4, ksum4)

    q3 = q_s.reshape(TBN, WS, C)
    kv_r3 = kv_r.reshape(TBN, HD, C)
    ks_r3 = ks_r.reshape(TBN, 1, C)
    hid = jnp.arange(C, dtype=jnp.int32) // HD
    bdmask = (hid[:, None] == hid[None, :]).astype(jnp.float32)
    out = pl.pallas_call(
        _attn_proj_kernel,
        grid=(TBN // WB,),
        in_specs=[pl.BlockSpec((WB, WS, C), lambda i: (i, 0, 0)),
                  pl.BlockSpec((WB, HD, C), lambda i: (i, 0, 0)),
                  pl.BlockSpec((WB, 1, C), lambda i: (i, 0, 0)),
                  pl.BlockSpec((C, C), lambda i: (0, 0)),
                  pl.BlockSpec((C, C), lambda i: (0, 0)),
                  pl.BlockSpec((1, C), lambda i: (0, 0))],
        out_specs=pl.BlockSpec((WB, WS, C), lambda i: (i, 0, 0)),
        out_shape=jax.ShapeDtypeStruct((TBN, WS, C), jnp.float32),
    )(q3, kv_r3, ks_r3, bdmask, W_proj.T, b_proj.reshape(1, C))

    out = out.reshape(T, B, wt, wh, ww, nt, nh, nd, C)
    out = jnp.transpose(out, (0, 1, 2, 5, 3, 6, 4, 7, 8))
    return out.reshape(T, B, Lt, Lh, Lw, C)


# speedup vs baseline: 2.7569x; 1.0977x over previous
import jax
import jax.numpy as jnp
from jax import lax
from jax.experimental import pallas as pl

C = 384
H = 8
HD = 48
NW = 64
WS = 32
TOPK = 4


CHW = 32


def _qkv_kvw_kernel(x_ref, w_ref, b_ref, q_ref, kvw_ref, ksum_ref, feat_ref):
    x = x_ref[...]
    acc = jnp.dot(x, w_ref[...], preferred_element_type=jnp.float32) + b_ref[...]
    s = (acc >= 2.0).astype(jnp.bfloat16)
    q_ref[...] = s[:, :C]
    k = s[:, C:2 * C]
    v = s[:, 2 * C:]
    for w in range(CHW):
        rows = slice(w * WS, (w + 1) * WS)
        kw = k[rows]
        vw = v[rows]
        kvf = lax.dot_general(kw, vw, (((0,), (0,)), ((), ())),
                              preferred_element_type=jnp.float32)
        kvf = kvf.astype(jnp.bfloat16)
        pieces = [lax.slice(kvf, (h * HD, h * HD), ((h + 1) * HD, (h + 1) * HD))
                  for h in range(H)]
        kvw_ref[w] = jnp.concatenate(pieces, axis=1)
        ksum_ref[w] = jnp.sum(kw.astype(jnp.float32), axis=0,
                              keepdims=True).astype(jnp.bfloat16)
        feat_ref[w] = jnp.sum(x[rows], axis=0, keepdims=True)


def _routing_kernel(xr_ref, m_ref):
    xs = xr_ref[:, 0]
    feat = jnp.sum(xs, axis=0) * (1.0 / WS)
    scores = lax.dot_general(
        feat, feat, (((1,), (1,)), ((), ())),
        preferred_element_type=jnp.float32) * (HD ** -0.5)
    iota = lax.broadcasted_iota(jnp.int32, (NW, NW), 1)
    mask = jnp.zeros((NW, NW), jnp.float32)
    s = scores
    for _ in range(TOPK):
        mx = jnp.max(s, axis=1, keepdims=True)
        cand = jnp.where(s >= mx, iota, jnp.int32(2 ** 30))
        mi = jnp.min(cand, axis=1, keepdims=True)
        sel = iota == mi
        mask = jnp.where(sel, 1.0, mask)
        s = jnp.where(sel, -jnp.inf, s)
    m_ref[0] = mask.astype(jnp.bfloat16)


WB = 8


def _route_kv_kernel(m_ref, kvw_ref, ksum_ref, kvr_ref, ksr_ref):
    m = m_ref[0]
    kvr_ref[0, 0] = jnp.dot(m, kvw_ref[0, 0],
                            preferred_element_type=jnp.float32).astype(jnp.bfloat16)
    ksr_ref[0, 0] = jnp.dot(m, ksum_ref[0, 0],
                            preferred_element_type=jnp.float32).astype(jnp.bfloat16)


def _attn_proj_kernel(q_ref, kvr_ref, ksr_ref, bdm_ref, wp_ref, bp_ref, o_ref):
    bdm = bdm_ref[...]
    rows = []
    for w in range(WB):
        q = q_ref[w]
        kvr = kvr_ref[w]
        ks = ksr_ref[w]
        bd = jnp.concatenate([kvr] * H, axis=0) * bdm
        numer = jnp.dot(q, bd, preferred_element_type=jnp.float32)
        drep = jnp.dot(q * ks, bdm, preferred_element_type=jnp.float32)
        rows.append(numer / (drep + 1e-6))
    o = jnp.concatenate(rows, axis=0)
    o = jnp.dot(o, wp_ref[...], preferred_element_type=jnp.float32) + bp_ref[...]
    for w in range(WB):
        o_ref[w] = o[w * WS:(w + 1) * WS, :]


def kernel(x, W_qkv, b_qkv, W_proj, b_proj):
    T, B, Lt, Lh, Lw, _ = x.shape
    wt, wh, ww = 4, 4, 4
    nt, nh, nd = Lt // wt, Lh // wh, Lw // ww
    TBN = T * B * NW
    NTOK = TBN * WS

    xw = x.reshape(T, B, wt, nt, wh, nh, ww, nd, C)
    xw = jnp.transpose(xw, (0, 1, 2, 4, 6, 3, 5, 7, 8))
    x_win = xw.reshape(T, B, NW, WS, C)
    x_tok = x_win.reshape(NTOK, C)

    Wt_qkv = W_qkv.T
    bq = b_qkv.reshape(1, 3 * C)

    nblk = TBN // CHW
    q_s, kvw, ksum, feats = pl.pallas_call(
        _qkv_kvw_kernel,
        grid=(nblk,),
        in_specs=[pl.BlockSpec((NTOK // nblk, C), lambda i: (i, 0)),
                  pl.BlockSpec((C, 3 * C), lambda i: (0, 0)),
                  pl.BlockSpec((1, 3 * C), lambda i: (0, 0))],
        out_specs=[pl.BlockSpec((NTOK // nblk, C), lambda i: (i, 0)),
                   pl.BlockSpec((CHW, HD, C), lambda i: (i, 0, 0)),
                   pl.BlockSpec((CHW, 1, C), lambda i: (i, 0, 0)),
                   pl.BlockSpec((CHW, 1, C), lambda i: (i, 0, 0))],
        out_shape=[jax.ShapeDtypeStruct((NTOK, C), jnp.bfloat16),
                   jax.ShapeDtypeStruct((TBN, HD, C), jnp.bfloat16),
                   jax.ShapeDtypeStruct((TBN, 1, C), jnp.bfloat16),
                   jax.ShapeDtypeStruct((TBN, 1, C), jnp.float32)],
    )(x_tok, Wt_qkv, bq)

    feats4 = feats.reshape(T, B, NW, C)
    mask = pl.pallas_call(
        _routing_kernel,
        grid=(B,),
        in_specs=[pl.BlockSpec((T, 1, NW, C), lambda b: (0, b, 0, 0))],
        out_specs=pl.BlockSpec((1, NW, NW), lambda b: (b, 0, 0)),
        out_shape=jax.ShapeDtypeStruct((B, NW, NW), jnp.bfloat16),
    )(feats4)

    kvw4 = kvw.reshape(T, B, NW, HD * C)
    ksum4 = ksum.reshape(T, B, NW, C)
    kv_r, ks_r = pl.pallas_call(
        _route_kv_kernel,
        grid=(T, B),
        in_specs=[pl.BlockSpec((1, NW, NW), lambda t, b: (b, 0, 0)),
                  pl.BlockSpec((1, 1, NW, HD * C), lambda t, b: (t, b, 0, 0)),
                  pl.BlockSpec((1, 1, NW, C), lambda t, b: (t, b, 0, 0))],
        out_specs=[pl.BlockSpec((1, 1, NW, HD * C), lambda t, b: (t, b, 0, 0)),
                   pl.BlockSpec((1, 1, NW, C), lambda t, b: (t, b, 0, 0))],
        out_shape=[jax.ShapeDtypeStruct((T, B, NW, HD * C), jnp.bfloat16),
                   jax.ShapeDtypeStruct((T, B, NW, C), jnp.bfloat16)],
    )(mask, kvw4, ksum4)

    q3 = q_s.reshape(TBN, WS, C)
    kv_r3 = kv_r.reshape(TBN, HD, C)
    ks_r3 = ks_r.reshape(TBN, 1, C)
    hid = jnp.arange(C, dtype=jnp.int32) // HD
    bdmask = (hid[:, None] == hid[None, :]).astype(jnp.bfloat16)
    out = pl.pallas_call(
        _attn_proj_kernel,
        grid=(TBN // WB,),
        in_specs=[pl.BlockSpec((WB, WS, C), lambda i: (i, 0, 0)),
                  pl.BlockSpec((WB, HD, C), lambda i: (i, 0, 0)),
                  pl.BlockSpec((WB, 1, C), lambda i: (i, 0, 0)),
                  pl.BlockSpec((C, C), lambda i: (0, 0)),
                  pl.BlockSpec((C, C), lambda i: (0, 0)),
                  pl.BlockSpec((1, C), lambda i: (0, 0))],
        out_specs=pl.BlockSpec((WB, WS, C), lambda i: (i, 0, 0)),
        out_shape=jax.ShapeDtypeStruct((TBN, WS, C), jnp.float32),
    )(q3, kv_r3, ks_r3, bdmask, W_proj.T, b_proj.reshape(1, C))

    out = out.reshape(T, B, wt, wh, ww, nt, nh, nd, C)
    out = jnp.transpose(out, (0, 1, 2, 5, 3, 6, 4, 7, 8))
    return out.reshape(T, B, Lt, Lh, Lw, C)


# speedup vs baseline: 3.4894x; 1.2657x over previous
import jax
import jax.numpy as jnp
from jax import lax
from jax.experimental import pallas as pl
from jax.experimental.pallas import tpu as pltpu

C = 384
H = 8
HD = 48
NW = 64
WS = 32
TOPK = 4

CHW = 32
WPAD = 64


def _qkv_kvw_kernel(x_ref, w_ref, b_ref, q_ref, kvw_ref, feat_ref):
    x = x_ref[...]
    acc = jnp.dot(x, w_ref[...], preferred_element_type=jnp.float32) + b_ref[...]
    s = (acc >= 2.0).astype(jnp.bfloat16)
    q_ref[...] = s[:, :C]
    k = s[:, C:2 * C]
    v = s[:, 2 * C:]
    for w in range(CHW):
        rows = slice(w * WS, (w + 1) * WS)
        kw = k[rows]
        vw = v[rows]
        kvf = lax.dot_general(kw, vw, (((0,), (0,)), ((), ())),
                              preferred_element_type=jnp.float32)
        kvf = kvf.astype(jnp.bfloat16)
        pieces = [lax.slice(kvf, (h * HD, h * HD), ((h + 1) * HD, (h + 1) * HD))
                  for h in range(H)]
        kvwp = jnp.concatenate(pieces, axis=1)
        ksum = jnp.sum(kw.astype(jnp.float32), axis=0,
                       keepdims=True).astype(jnp.bfloat16)
        pad = jnp.zeros((WPAD - HD - 1, C), jnp.bfloat16)
        kvw_ref[w] = jnp.concatenate([kvwp, ksum, pad], axis=0)
        feat_ref[w] = jnp.sum(x[rows], axis=0, keepdims=True)


def _routing_kernel(xr_ref, idx_ref):
    xs = xr_ref[:, 0]
    feat = jnp.sum(xs, axis=0) * (1.0 / WS)
    scores = lax.dot_general(
        feat, feat, (((1,), (1,)), ((), ())),
        preferred_element_type=jnp.float32) * (HD ** -0.5)
    iota = lax.broadcasted_iota(jnp.int32, (NW, NW), 1)
    s = scores
    mis = []
    for _ in range(TOPK):
        mx = jnp.max(s, axis=1, keepdims=True)
        cand = jnp.where(s >= mx, iota, jnp.int32(2 ** 30))
        mi = jnp.min(cand, axis=1, keepdims=True)
        mis.append(mi)
        sel = iota == mi
        s = jnp.where(sel, -jnp.inf, s)
    idx_ref[0] = jnp.concatenate(mis, axis=1)


def _attn_proj_kernel(idx_ref, q_ref, kvw_ref, bdm_ref, wp_ref,
                      bp_ref, o_ref):
    b = pl.program_id(1)
    bdm = bdm_ref[...]
    rows = []
    for w in range(NW):
        i0 = idx_ref[b, w, 0]
        i1 = idx_ref[b, w, 1]
        i2 = idx_ref[b, w, 2]
        i3 = idx_ref[b, w, 3]
        blk = (kvw_ref[0, pl.ds(pl.multiple_of(i0 * WPAD, WPAD), WPAD), :]
               + kvw_ref[0, pl.ds(pl.multiple_of(i1 * WPAD, WPAD), WPAD), :]
               + kvw_ref[0, pl.ds(pl.multiple_of(i2 * WPAD, WPAD), WPAD), :]
               + kvw_ref[0, pl.ds(pl.multiple_of(i3 * WPAD, WPAD), WPAD), :])
        kvr = blk[:HD]
        ks = blk[HD:HD + 1]
        q = q_ref[0, pl.ds(w * WS, WS), :]
        bd = jnp.concatenate([kvr] * H, axis=0) * bdm
        numer = jnp.dot(q, bd, preferred_element_type=jnp.float32)
        drep = jnp.dot(q * ks, bdm, preferred_element_type=jnp.float32)
        rows.append(numer / (drep + 1e-6))
    o = jnp.concatenate(rows, axis=0)
    o_ref[0] = jnp.dot(o, wp_ref[...], preferred_element_type=jnp.float32) \
        + bp_ref[...]


def kernel(x, W_qkv, b_qkv, W_proj, b_proj):
    T, B, Lt, Lh, Lw, _ = x.shape
    wt, wh, ww = 4, 4, 4
    nt, nh, nd = Lt // wt, Lh // wh, Lw // ww
    TBN = T * B * NW
    NTOK = TBN * WS

    xw = x.reshape(T, B, wt, nt, wh, nh, ww, nd, C)
    xw = jnp.transpose(xw, (0, 1, 2, 4, 6, 3, 5, 7, 8))
    x_win = xw.reshape(T, B, NW, WS, C)
    x_tok = x_win.reshape(NTOK, C)

    Wt_qkv = W_qkv.T
    bq = b_qkv.reshape(1, 3 * C)

    nblk = TBN // CHW
    q_s, kvw, feats = pl.pallas_call(
        _qkv_kvw_kernel,
        grid=(nblk,),
        in_specs=[pl.BlockSpec((NTOK // nblk, C), lambda i: (i, 0)),
                  pl.BlockSpec((C, 3 * C), lambda i: (0, 0)),
                  pl.BlockSpec((1, 3 * C), lambda i: (0, 0))],
        out_specs=[pl.BlockSpec((NTOK // nblk, C), lambda i: (i, 0)),
                   pl.BlockSpec((CHW, WPAD, C), lambda i: (i, 0, 0)),
                   pl.BlockSpec((CHW, 1, C), lambda i: (i, 0, 0))],
        out_shape=[jax.ShapeDtypeStruct((NTOK, C), jnp.bfloat16),
                   jax.ShapeDtypeStruct((TBN, WPAD, C), jnp.bfloat16),
                   jax.ShapeDtypeStruct((TBN, 1, C), jnp.float32)],
    )(x_tok, Wt_qkv, bq)

    feats4 = feats.reshape(T, B, NW, C)
    idx = pl.pallas_call(
        _routing_kernel,
        grid=(B,),
        in_specs=[pl.BlockSpec((T, 1, NW, C), lambda b: (0, b, 0, 0))],
        out_specs=pl.BlockSpec((1, NW, TOPK), lambda b: (b, 0, 0)),
        out_shape=jax.ShapeDtypeStruct((B, NW, TOPK), jnp.int32),
    )(feats4)

    q3 = q_s.reshape(T * B, NW * WS, C)
    kvw3 = kvw.reshape(T * B, NW * WPAD, C)
    hid = jnp.arange(C, dtype=jnp.int32) // HD
    bdmask = (hid[:, None] == hid[None, :]).astype(jnp.bfloat16)
    out = pl.pallas_call(
        _attn_proj_kernel,
        grid=(T, B),
        in_specs=[pl.BlockSpec(memory_space=pltpu.SMEM),
                  pl.BlockSpec((1, NW * WS, C), lambda t, b: (t * B + b, 0, 0)),
                  pl.BlockSpec((1, NW * WPAD, C), lambda t, b: (t * B + b, 0, 0)),
                  pl.BlockSpec((C, C), lambda t, b: (0, 0)),
                  pl.BlockSpec((C, C), lambda t, b: (0, 0)),
                  pl.BlockSpec((1, C), lambda t, b: (0, 0))],
        out_specs=pl.BlockSpec((1, NW * WS, C), lambda t, b: (t * B + b, 0, 0)),
        out_shape=jax.ShapeDtypeStruct((T * B, NW * WS, C), jnp.float32),
    )(idx, q3, kvw3, bdmask, W_proj.T, b_proj.reshape(1, C))

    out = out.reshape(T, B, wt, wh, ww, nt, nh, nd, C)
    out = jnp.transpose(out, (0, 1, 2, 5, 3, 6, 4, 7, 8))
    return out.reshape(T, B, Lt, Lh, Lw, C)


# speedup vs baseline: 3.6375x; 1.0424x over previous
import jax
import jax.numpy as jnp
from jax import lax
from jax.experimental import pallas as pl
from jax.experimental.pallas import tpu as pltpu

C = 384
H = 8
HD = 48
NW = 64
WS = 32
TOPK = 4

CHW = 32
WPAD = 64


def _qkv_kvw_kernel(x_ref, w_ref, b_ref, q_ref, kvw_ref, feat_ref):
    x = x_ref[...]
    acc = jnp.dot(x, w_ref[...], preferred_element_type=jnp.float32) + b_ref[...]
    s = (acc >= 2.0).astype(jnp.bfloat16)
    q_ref[...] = s[:, :C]
    k = s[:, C:2 * C]
    v = s[:, 2 * C:]
    for w in range(CHW):
        rows = slice(w * WS, (w + 1) * WS)
        kw = k[rows]
        vw = v[rows]
        kvf = lax.dot_general(kw, vw, (((0,), (0,)), ((), ())),
                              preferred_element_type=jnp.float32)
        kvf = kvf.astype(jnp.bfloat16)
        pieces = [lax.slice(kvf, (h * HD, h * HD), ((h + 1) * HD, (h + 1) * HD))
                  for h in range(H)]
        kvwp = jnp.concatenate(pieces, axis=1)
        ksum = jnp.sum(kw.astype(jnp.float32), axis=0,
                       keepdims=True).astype(jnp.bfloat16)
        pad = jnp.zeros((WPAD - HD - 1, C), jnp.bfloat16)
        kvw_ref[w] = jnp.concatenate([kvwp, ksum, pad], axis=0)
        feat_ref[w] = jnp.sum(x[rows], axis=0, keepdims=True)


def _routing_kernel(xr_ref, idx_ref):
    xs = xr_ref[:, 0]
    feat = jnp.sum(xs, axis=0) * (1.0 / WS)
    scores = lax.dot_general(
        feat, feat, (((1,), (1,)), ((), ())),
        preferred_element_type=jnp.float32) * (HD ** -0.5)
    iota = lax.broadcasted_iota(jnp.int32, (NW, NW), 1)
    s = scores
    mis = []
    for _ in range(TOPK):
        mx = jnp.max(s, axis=1, keepdims=True)
        cand = jnp.where(s >= mx, iota, jnp.int32(2 ** 30))
        mi = jnp.min(cand, axis=1, keepdims=True)
        mis.append(mi)
        sel = iota == mi
        s = jnp.where(sel, -jnp.inf, s)
    idx_ref[0] = jnp.concatenate(mis, axis=1)


def _attn_proj_kernel(idx_ref, q_ref, kvw_ref, bdm_ref, wp_ref,
                      bp_ref, o_ref):
    b = pl.program_id(1)
    bdm = bdm_ref[...]
    numers = []
    qks = []
    for w in range(NW):
        i0 = idx_ref[b, w, 0]
        i1 = idx_ref[b, w, 1]
        i2 = idx_ref[b, w, 2]
        i3 = idx_ref[b, w, 3]
        blk = (kvw_ref[0, pl.ds(pl.multiple_of(i0 * WPAD, WPAD), WPAD), :]
               + kvw_ref[0, pl.ds(pl.multiple_of(i1 * WPAD, WPAD), WPAD), :]
               + kvw_ref[0, pl.ds(pl.multiple_of(i2 * WPAD, WPAD), WPAD), :]
               + kvw_ref[0, pl.ds(pl.multiple_of(i3 * WPAD, WPAD), WPAD), :])
        kvr = blk[:HD]
        ks = blk[HD:HD + 1]
        q = q_ref[0, pl.ds(w * WS, WS), :]
        bd = jnp.concatenate([kvr] * H, axis=0) * bdm
        numers.append(jnp.dot(q, bd, preferred_element_type=jnp.float32))
        qks.append(q * ks)
    numer = jnp.concatenate(numers, axis=0)
    drep = jnp.dot(jnp.concatenate(qks, axis=0), bdm,
                   preferred_element_type=jnp.float32)
    o = numer / (drep + 1e-6)
    o_ref[0] = jnp.dot(o, wp_ref[...], preferred_element_type=jnp.float32) \
        + bp_ref[...]


def kernel(x, W_qkv, b_qkv, W_proj, b_proj):
    T, B, Lt, Lh, Lw, _ = x.shape
    wt, wh, ww = 4, 4, 4
    nt, nh, nd = Lt // wt, Lh // wh, Lw // ww
    TBN = T * B * NW
    NTOK = TBN * WS

    xw = x.reshape(T, B, wt, nt, wh, nh, ww, nd, C)
    xw = jnp.transpose(xw, (0, 1, 2, 4, 6, 3, 5, 7, 8))
    x_win = xw.reshape(T, B, NW, WS, C)
    x_tok = x_win.reshape(NTOK, C)

    Wt_qkv = W_qkv.T
    bq = b_qkv.reshape(1, 3 * C)

    nblk = TBN // CHW
    q_s, kvw, feats = pl.pallas_call(
        _qkv_kvw_kernel,
        grid=(nblk,),
        in_specs=[pl.BlockSpec((NTOK // nblk, C), lambda i: (i, 0)),
                  pl.BlockSpec((C, 3 * C), lambda i: (0, 0)),
                  pl.BlockSpec((1, 3 * C), lambda i: (0, 0))],
        out_specs=[pl.BlockSpec((NTOK // nblk, C), lambda i: (i, 0)),
                   pl.BlockSpec((CHW, WPAD, C), lambda i: (i, 0, 0)),
                   pl.BlockSpec((CHW, 1, C), lambda i: (i, 0, 0))],
        out_shape=[jax.ShapeDtypeStruct((NTOK, C), jnp.bfloat16),
                   jax.ShapeDtypeStruct((TBN, WPAD, C), jnp.bfloat16),
                   jax.ShapeDtypeStruct((TBN, 1, C), jnp.float32)],
    )(x_tok, Wt_qkv, bq)

    feats4 = feats.reshape(T, B, NW, C)
    idx = pl.pallas_call(
        _routing_kernel,
        grid=(B,),
        in_specs=[pl.BlockSpec((T, 1, NW, C), lambda b: (0, b, 0, 0))],
        out_specs=pl.BlockSpec((1, NW, TOPK), lambda b: (b, 0, 0)),
        out_shape=jax.ShapeDtypeStruct((B, NW, TOPK), jnp.int32),
    )(feats4)

    q3 = q_s.reshape(T * B, NW * WS, C)
    kvw3 = kvw.reshape(T * B, NW * WPAD, C)
    hid = jnp.arange(C, dtype=jnp.int32) // HD
    bdmask = (hid[:, None] == hid[None, :]).astype(jnp.bfloat16)
    out = pl.pallas_call(
        _attn_proj_kernel,
        grid=(T, B),
        in_specs=[pl.BlockSpec(memory_space=pltpu.SMEM),
                  pl.BlockSpec((1, NW * WS, C), lambda t, b: (t * B + b, 0, 0)),
                  pl.BlockSpec((1, NW * WPAD, C), lambda t, b: (t * B + b, 0, 0)),
                  pl.BlockSpec((C, C), lambda t, b: (0, 0)),
                  pl.BlockSpec((C, C), lambda t, b: (0, 0)),
                  pl.BlockSpec((1, C), lambda t, b: (0, 0))],
        out_specs=pl.BlockSpec((1, NW * WS, C), lambda t, b: (t * B + b, 0, 0)),
        out_shape=jax.ShapeDtypeStruct((T * B, NW * WS, C), jnp.float32),
    )(idx, q3, kvw3, bdmask, W_proj.T, b_proj.reshape(1, C))

    out = out.reshape(T, B, wt, wh, ww, nt, nh, nd, C)
    out = jnp.transpose(out, (0, 1, 2, 5, 3, 6, 4, 7, 8))
    return out.reshape(T, B, Lt, Lh, Lw, C)
